# baseline (speedup 1.0000x reference)
import functools

import jax
import jax.numpy as jnp
from jax import lax
from jax.experimental import pallas as pl
from jax.experimental.pallas import tpu as pltpu
from jax.experimental.pallas import tpu_sc as plsc

B = 1024
D = 64
NUSERS = 32
F = 8
NF = NUSERS * F
L = 16

_W1_OFF = 0
_B1_OFF = _W1_OFF + 2048
_W2_OFF = _B1_OFF + 16
_B2_OFF = _W2_OFF + 16
_WP1_OFF = _B2_OFF + 16
_BP1_OFF = _WP1_OFF + 3072
_WP2_OFF = _BP1_OFF + 16
_BP2_OFF = _WP2_OFF + 16
_WLEN = _BP2_OFF + 16


def _sc_forward(item_table, user_table, follow_table, item_idx, follow_idx,
                user_idx, wflat):
  info = plsc.get_sparse_core_info()
  nc, ns = info.num_cores, info.num_subcores
  nw = nc * ns
  bi = B // nw
  upt = NUSERS // ns
  mesh = plsc.VectorSubcoreMesh(core_axis_name="c", subcore_axis_name="s")

  @functools.partial(
      pl.kernel,
      mesh=mesh,
      compiler_params=pltpu.CompilerParams(needs_layout_passes=False),
      out_type=jax.ShapeDtypeStruct((B,), jnp.float32),
      scratch_types=[
          pltpu.VMEM((bi,), jnp.int32),
          pltpu.VMEM((bi,), jnp.int32),
          pltpu.VMEM((L,), jnp.int32),
          pltpu.VMEM((bi, D), jnp.float32),
          pltpu.VMEM((upt * F, D), jnp.float32),
          pltpu.VMEM((upt, D), jnp.float32),
          pltpu.VMEM((_WLEN,), jnp.float32),
          pltpu.VMEM((upt, D), jnp.float32),
          pltpu.VMEM((NUSERS, D), jnp.float32),
          pltpu.VMEM((bi,), jnp.float32),
          pltpu.VMEM_SHARED((NUSERS, D), jnp.float32),
          pltpu.SemaphoreType.DMA,
          pltpu.SemaphoreType.DMA,
      ],
  )
  def k(items_hbm, users_hbm, follows_hbm, iidx_hbm, fidx_hbm, uidx_hbm,
        w_hbm, y_hbm, iidx_v, uidx_v, fidx_v, irows_v, frows_v, ue_v, w_v,
        uown_v, uall_v, y_v, ushared, sem, isem):
    t = lax.axis_index("s")
    c = lax.axis_index("c")
    wid = t * nc + c
    ib = wid * bi

    pltpu.sync_copy(iidx_hbm.at[pl.ds(ib, bi)], iidx_v)
    pltpu.sync_copy(uidx_hbm.at[pl.ds(ib, bi)], uidx_v)
    pltpu.sync_copy(fidx_hbm.at[pl.ds(t * (upt * F), upt * F)],
                    fidx_v.at[pl.ds(0, upt * F)])
    pltpu.async_copy(w_hbm, w_v, sem)

    for cb in range(bi // L):
      ivec = iidx_v[pl.ds(cb * L, L)]
      for l in range(L):
        pltpu.async_copy(items_hbm.at[pl.ds(ivec[l], 1)],
                         irows_v.at[pl.ds(cb * L + l, 1)], isem)

    fvec = fidx_v[...]
    for j in range(upt * F):
      pltpu.async_copy(follows_hbm.at[pl.ds(fvec[j], 1)],
                       frows_v.at[pl.ds(j, 1)], sem)
    for m in range(upt):
      pltpu.async_copy(users_hbm.at[pl.ds(t * upt + m, 1)],
                       ue_v.at[pl.ds(m, 1)], sem)
    pltpu.make_async_copy(w_hbm, w_v, sem).wait()
    for j in range(upt * F):
      pltpu.make_async_copy(follows_hbm.at[pl.ds(0, 1)],
                            frows_v.at[pl.ds(j, 1)], sem).wait()
    for m in range(upt):
      pltpu.make_async_copy(users_hbm.at[pl.ds(0, 1)],
                            ue_v.at[pl.ds(m, 1)], sem).wait()

    lanes = lax.iota(jnp.int32, L)
    lane_lt_f = lanes < F
    b1v = w_v[pl.ds(_B1_OFF, L)]
    w2v = w_v[pl.ds(_W2_OFF, L)]

    for m in range(upt):
      uew = b1v
      for cc in range(D // L):
        uev = ue_v[m, pl.ds(cc * L, L)]
        for l in range(L):
          d = cc * L + l
          uew = uew + uev[l] * w_v[pl.ds(_W1_OFF + 16 * (D + d), L)]
      svec = jnp.zeros((L,), jnp.float32)
      for j in range(F):
        h = uew
        for cc in range(D // L):
          fev = frows_v[m * F + j, pl.ds(cc * L, L)]
          for l in range(L):
            d = cc * L + l
            h = h + fev[l] * w_v[pl.ds(_W1_OFF + 16 * d, L)]
        h = jnp.maximum(h, 0.0)
        sj = jnp.sum(h * w2v) + w_v[pl.ds(_B2_OFF, L)][0]
        svec = svec + sj * (lanes == j).astype(jnp.float32)
      svec = jnp.where(lane_lt_f, svec, -1e30)
      svec = svec - jnp.max(svec)
      e = jnp.where(lane_lt_f, jnp.exp(svec), 0.0)
      p = e / jnp.sum(e)
      for cc in range(D // L):
        acc = ue_v[m, pl.ds(cc * L, L)]
        for j in range(F):
          acc = acc + p[j] * frows_v[m * F + j, pl.ds(cc * L, L)]
        uown_v[m, pl.ds(cc * L, L)] = acc

    pltpu.sync_copy(uown_v, ushared.at[pl.ds(t * upt, upt)])
    plsc.subcore_barrier()
    pltpu.sync_copy(ushared, uall_v)

    for j in range(bi):
      pltpu.make_async_copy(items_hbm.at[pl.ds(0, 1)],
                            irows_v.at[pl.ds(j, 1)], isem).wait()

    bp1v = w_v[pl.ds(_BP1_OFF, L)]
    wp2v = w_v[pl.ds(_WP2_OFF, L)]
    bp2s = w_v[pl.ds(_BP2_OFF, L)][0]

    for blk in range(bi // L):
      rows = blk * L + lanes
      uid_vec = uidx_v[pl.ds(blk * L, L)]

      def dbody(d, accs):
        iecol = plsc.load_gather(irows_v, [rows, jnp.full((L,), d)])
        ubcol = plsc.load_gather(uall_v, [uid_vec, jnp.full((L,), d)])
        elcol = iecol * ubcol
        wa = w_v[pl.ds(_WP1_OFF + 16 * d, L)]
        wb = w_v[pl.ds(_WP1_OFF + 16 * (D + d), L)]
        wc = w_v[pl.ds(_WP1_OFF + 16 * (2 * D + d), L)]
        return tuple(
            accs[kk] + elcol * wa[kk] + ubcol * wb[kk] + iecol * wc[kk]
            for kk in range(F))

      accs0 = tuple(jnp.full((L,), bp1v[kk]) for kk in range(F))
      accs = lax.fori_loop(0, D, dbody, accs0)
      z = jnp.zeros((L,), jnp.float32)
      for kk in range(F):
        z = z + jnp.maximum(accs[kk], 0.0) * wp2v[kk]
      y_v[pl.ds(blk * L, L)] = 1.0 / (1.0 + jnp.exp(-(z + bp2s)))

    pltpu.sync_copy(y_v, y_hbm.at[pl.ds(ib, bi)])

  return k(item_table, user_table, follow_table, item_idx, follow_idx,
           user_idx, wflat)


def kernel(user_inputs, item_inputs, group_inputs, follows_all, user_table,
           item_table, follow_table, W1, b1, W2, b2, Wp1, bp1, Wp2, bp2):
  del group_inputs
  item_idx = item_inputs.astype(jnp.int32)
  follow_idx = follows_all.reshape(NF).astype(jnp.int32)
  user_idx = user_inputs.astype(jnp.int32)
  wflat = jnp.concatenate([
      W1.reshape(-1),
      b1,
      W2.reshape(-1),
      jnp.pad(b2, (0, 15)),
      jnp.pad(Wp1, ((0, 0), (0, 8))).reshape(-1),
      jnp.pad(bp1, (0, 8)),
      jnp.pad(Wp2[:, 0], (0, 8)),
      jnp.pad(bp2, (0, 15)),
  ])
  y = _sc_forward(item_table, user_table, follow_table, item_idx, follow_idx,
                  user_idx, wflat)
  return y.reshape(B, 1)
